# Initial kernel scaffold; baseline (speedup 1.0000x reference)
import jax
import jax.numpy as jnp
from jax.experimental import pallas as pl


def kernel(clinical, mel, edge_index, W_mel, b_mel, W_cat, b_cat, W1l, b1, W1r, W2l, b2, W2r):
    raise NotImplementedError("write your pallas kernel here")


# speedup vs baseline: 5.4488x; 5.4488x over previous
import functools

import jax
import jax.numpy as jnp
from jax import lax
from jax.experimental import pallas as pl
from jax.experimental.pallas import tpu as pltpu
from jax.experimental.pallas import tpu_sc as plsc

N = 10000
E = 320000
CLIN = 64
MEL = 128
HID = 128
NC = 4
D1 = 144
D2 = 16
BN = 1000
GRID = N // BN

_PREC = lax.Precision.HIGHEST


def _tc1_body(clin_ref, mel_ref, wm_ref, bm_ref, wc1_ref, wc2_ref, bc_ref,
              out_ref):
    mel_h = jnp.maximum(
        jnp.dot(mel_ref[...], wm_ref[...], precision=_PREC,
                preferred_element_type=jnp.float32) + bm_ref[...][None, :],
        0.0)
    xb = jnp.maximum(
        jnp.dot(clin_ref[...], wc1_ref[...], precision=_PREC,
                preferred_element_type=jnp.float32)
        + jnp.dot(mel_h, wc2_ref[...], precision=_PREC,
                  preferred_element_type=jnp.float32)
        + bc_ref[...][None, :],
        0.0)
    aug = (lax.broadcasted_iota(jnp.int32, (BN, D1 - HID), 1) == 0)
    out_ref[:, :HID] = xb
    out_ref[:, HID:] = aug.astype(jnp.float32)


def _tc2_body(agg_ref, xaug_ref, w1l_ref, b1_ref, w1r_ref, w2lp_ref, w2r_ref,
              b2_ref, p2_ref, r2_ref):
    aggs = agg_ref[0] + agg_ref[1]
    cnt = jnp.maximum(aggs[:, HID:HID + 1], 1.0)
    cinv = 1.0 / cnt
    mean1 = aggs[:, :HID] * cinv
    h1 = jnp.maximum(
        jnp.dot(mean1, w1l_ref[...], precision=_PREC,
                preferred_element_type=jnp.float32)
        + jnp.dot(xaug_ref[:, :HID], w1r_ref[...], precision=_PREC,
                  preferred_element_type=jnp.float32)
        + b1_ref[...][None, :],
        0.0)
    p2_ref[...] = jnp.dot(h1, w2lp_ref[...], precision=_PREC,
                          preferred_element_type=jnp.float32)
    r2 = jnp.dot(h1, w2r_ref[...], precision=_PREC,
                 preferred_element_type=jnp.float32) + b2_ref[...][None, :]
    r2_ref[...] = jnp.concatenate(
        [r2, cinv, jnp.zeros((BN, D2 - NC - 1), jnp.float32)], axis=1)


def _tc3_body(agg2_ref, r2_ref, out_ref):
    a = agg2_ref[0] + agg2_ref[1]
    out_ref[...] = a[:, :NC] * r2_ref[:, NC:NC + 1] + r2_ref[:, :NC]


def _full(shape):
    nd = len(shape)
    return pl.BlockSpec(shape, lambda i: (0,) * nd)


_tc1 = pl.pallas_call(
    _tc1_body,
    grid=(GRID,),
    in_specs=[
        pl.BlockSpec((BN, CLIN), lambda i: (i, 0)),
        pl.BlockSpec((BN, MEL), lambda i: (i, 0)),
        _full((MEL, HID)),
        _full((HID,)),
        _full((CLIN, HID)),
        _full((HID, HID)),
        _full((HID,)),
    ],
    out_specs=pl.BlockSpec((BN, D1), lambda i: (i, 0)),
    out_shape=jax.ShapeDtypeStruct((N, D1), jnp.float32),
)

_tc2 = pl.pallas_call(
    _tc2_body,
    grid=(GRID,),
    in_specs=[
        pl.BlockSpec((2, BN, D1), lambda i: (0, i, 0)),
        pl.BlockSpec((BN, D1), lambda i: (i, 0)),
        _full((HID, HID)),
        _full((HID,)),
        _full((HID, HID)),
        _full((HID, D2)),
        _full((HID, NC)),
        _full((NC,)),
    ],
    out_specs=[
        pl.BlockSpec((BN, D2), lambda i: (i, 0)),
        pl.BlockSpec((BN, D2), lambda i: (i, 0)),
    ],
    out_shape=[
        jax.ShapeDtypeStruct((N, D2), jnp.float32),
        jax.ShapeDtypeStruct((N, D2), jnp.float32),
    ],
)

_tc3 = pl.pallas_call(
    _tc3_body,
    grid=(GRID,),
    in_specs=[
        pl.BlockSpec((2, BN, D2), lambda i: (0, i, 0)),
        pl.BlockSpec((BN, D2), lambda i: (i, 0)),
    ],
    out_specs=pl.BlockSpec((BN, NC), lambda i: (i, 0)),
    out_shape=jax.ShapeDtypeStruct((N, NC), jnp.float32),
)


@functools.lru_cache(maxsize=None)
def _make_sc_agg(d, name):
    info = plsc.get_sparse_core_info()
    ncores, nsub = info.num_cores, info.num_subcores
    nw = ncores * nsub
    ept = E // nw
    K = 80
    nchunk = ept // K
    ZR = 125
    zchunks_per_tile = N // ZR // nsub
    rows_per_tile = N // nsub
    mesh = plsc.VectorSubcoreMesh(core_axis_name="c", subcore_axis_name="s")

    @functools.partial(
        pl.kernel,
        out_type=jax.ShapeDtypeStruct((ncores, N, d), jnp.float32),
        mesh=mesh,
        scratch_types=[
            pltpu.VMEM((K,), jnp.int32),
            pltpu.VMEM((K,), jnp.int32),
            pltpu.VMEM((K, d), jnp.float32),
            pltpu.VMEM((ZR, d), jnp.float32),
            pltpu.VMEM_SHARED((N, d), jnp.float32),
            pltpu.SemaphoreType.DMA,
        ],
        compiler_params=pltpu.CompilerParams(use_tc_tiling_on_sc=False),
        name=name,
    )
    def agg_kernel(x_hbm, src_hbm, dst_hbm, out_hbm,
                   srcb, dstb, rows, zbuf, agg_sh, gsem):
        c = lax.axis_index("c")
        s = lax.axis_index("s")
        wid = s * ncores + c

        def zfill_row(i, _):
            def zfill_col(j, _):
                zbuf[i, pl.ds(j * 16, 16)] = jnp.zeros((16,), jnp.float32)
                return ()
            return lax.fori_loop(0, d // 16, zfill_col, ())
        lax.fori_loop(0, ZR, zfill_row, ())

        def zero_chunk(j, _):
            r0 = (s * zchunks_per_tile + j) * ZR
            pltpu.sync_copy(zbuf, agg_sh.at[pl.ds(r0, ZR)])
            return ()
        lax.fori_loop(0, zchunks_per_tile, zero_chunk, ())
        plsc.subcore_barrier()

        def edge_chunk(j, _):
            e0 = wid * ept + j * K
            pltpu.sync_copy(src_hbm.at[pl.ds(e0, K)], srcb)
            pltpu.sync_copy(dst_hbm.at[pl.ds(e0, K)], dstb)
            pltpu.async_copy(x_hbm.at[srcb], rows, gsem).wait()
            pltpu.sync_copy(rows, agg_sh.at[dstb], add=True)
            return ()
        lax.fori_loop(0, nchunk, edge_chunk, ())
        plsc.subcore_barrier()

        r0 = s * rows_per_tile
        pltpu.sync_copy(agg_sh.at[pl.ds(r0, rows_per_tile)],
                        out_hbm.at[c].at[pl.ds(r0, rows_per_tile)])

    return agg_kernel


def kernel(clinical, mel, edge_index, W_mel, b_mel, W_cat, b_cat,
           W1l, b1, W1r, W2l, b2, W2r):
    src = edge_index[0]
    dst = edge_index[1]
    w2lp = jnp.pad(W2l, ((0, 0), (0, D2 - NC)))

    x_aug = _tc1(clinical, mel, W_mel, b_mel, W_cat[:CLIN], W_cat[CLIN:],
                 b_cat)
    agg1 = _make_sc_agg(D1, "sc_agg_l1")(x_aug, src, dst)
    p2, r2 = _tc2(agg1, x_aug, W1l, b1, W1r, w2lp, W2r, b2)
    agg2 = _make_sc_agg(D2, "sc_agg_l2")(p2, src, dst)
    return _tc3(agg2, r2)


# speedup vs baseline: 8.6253x; 1.5830x over previous
import functools

import jax
import jax.numpy as jnp
from jax import lax
from jax.experimental import pallas as pl
from jax.experimental.pallas import tpu as pltpu
from jax.experimental.pallas import tpu_sc as plsc

N = 10000
E = 320000
CLIN = 64
MEL = 128
HID = 128
NC = 4
D1 = 144
D2 = 16
BN = 1000
GRID = N // BN

_PREC = lax.Precision.HIGHEST


def _tc1_body(clin_ref, mel_ref, wm_ref, bm_ref, wc1_ref, wc2_ref, bc_ref,
              out_ref):
    mel_h = jnp.maximum(
        jnp.dot(mel_ref[...], wm_ref[...], precision=_PREC,
                preferred_element_type=jnp.float32) + bm_ref[...][None, :],
        0.0)
    xb = jnp.maximum(
        jnp.dot(clin_ref[...], wc1_ref[...], precision=_PREC,
                preferred_element_type=jnp.float32)
        + jnp.dot(mel_h, wc2_ref[...], precision=_PREC,
                  preferred_element_type=jnp.float32)
        + bc_ref[...][None, :],
        0.0)
    aug = (lax.broadcasted_iota(jnp.int32, (BN, D1 - HID), 1) == 0)
    out_ref[:, :HID] = xb
    out_ref[:, HID:] = aug.astype(jnp.float32)


def _tc2_body(agg_ref, xaug_ref, w1l_ref, b1_ref, w1r_ref, w2lp_ref, w2r_ref,
              b2_ref, p2_ref, r2_ref):
    aggs = agg_ref[0] + agg_ref[1]
    cnt = jnp.maximum(aggs[:, HID:HID + 1], 1.0)
    cinv = 1.0 / cnt
    mean1 = aggs[:, :HID] * cinv
    h1 = jnp.maximum(
        jnp.dot(mean1, w1l_ref[...], precision=_PREC,
                preferred_element_type=jnp.float32)
        + jnp.dot(xaug_ref[:, :HID], w1r_ref[...], precision=_PREC,
                  preferred_element_type=jnp.float32)
        + b1_ref[...][None, :],
        0.0)
    p2_ref[...] = jnp.dot(h1, w2lp_ref[...], precision=_PREC,
                          preferred_element_type=jnp.float32)
    r2 = jnp.dot(h1, w2r_ref[...], precision=_PREC,
                 preferred_element_type=jnp.float32) + b2_ref[...][None, :]
    r2_ref[...] = jnp.concatenate(
        [r2, cinv, jnp.zeros((BN, D2 - NC - 1), jnp.float32)], axis=1)


def _tc3_body(agg2_ref, r2_ref, out_ref):
    a = agg2_ref[0] + agg2_ref[1]
    out_ref[...] = a[:, :NC] * r2_ref[:, NC:NC + 1] + r2_ref[:, :NC]


def _full(shape):
    nd = len(shape)
    return pl.BlockSpec(shape, lambda i: (0,) * nd)


_tc1 = pl.pallas_call(
    _tc1_body,
    grid=(GRID,),
    in_specs=[
        pl.BlockSpec((BN, CLIN), lambda i: (i, 0)),
        pl.BlockSpec((BN, MEL), lambda i: (i, 0)),
        _full((MEL, HID)),
        _full((HID,)),
        _full((CLIN, HID)),
        _full((HID, HID)),
        _full((HID,)),
    ],
    out_specs=pl.BlockSpec((BN, D1), lambda i: (i, 0)),
    out_shape=jax.ShapeDtypeStruct((N, D1), jnp.float32),
)

_tc2 = pl.pallas_call(
    _tc2_body,
    grid=(GRID,),
    in_specs=[
        pl.BlockSpec((2, BN, D1), lambda i: (0, i, 0)),
        pl.BlockSpec((BN, D1), lambda i: (i, 0)),
        _full((HID, HID)),
        _full((HID,)),
        _full((HID, HID)),
        _full((HID, D2)),
        _full((HID, NC)),
        _full((NC,)),
    ],
    out_specs=[
        pl.BlockSpec((BN, D2), lambda i: (i, 0)),
        pl.BlockSpec((BN, D2), lambda i: (i, 0)),
    ],
    out_shape=[
        jax.ShapeDtypeStruct((N, D2), jnp.float32),
        jax.ShapeDtypeStruct((N, D2), jnp.float32),
    ],
)

_tc3 = pl.pallas_call(
    _tc3_body,
    grid=(GRID,),
    in_specs=[
        pl.BlockSpec((2, BN, D2), lambda i: (0, i, 0)),
        pl.BlockSpec((BN, D2), lambda i: (i, 0)),
    ],
    out_specs=pl.BlockSpec((BN, NC), lambda i: (i, 0)),
    out_shape=jax.ShapeDtypeStruct((N, NC), jnp.float32),
)


@functools.lru_cache(maxsize=None)
def _make_sc_agg(d, name):
    info = plsc.get_sparse_core_info()
    ncores, nsub = info.num_cores, info.num_subcores
    nw = ncores * nsub
    ept = E // nw
    CR = 125
    nchunk = ept // CR
    npair = nchunk // 2
    rows_per_tile = N // nsub
    mesh = plsc.VectorSubcoreMesh(core_axis_name="c", subcore_axis_name="s")

    @functools.partial(
        pl.kernel,
        out_type=jax.ShapeDtypeStruct((ncores, N, d), jnp.float32),
        mesh=mesh,
        scratch_types=[
            pltpu.VMEM((2, CR), jnp.int32),
            pltpu.VMEM((2, CR), jnp.int32),
            pltpu.VMEM((CR, d), jnp.float32),
            pltpu.VMEM((CR, d), jnp.float32),
            pltpu.VMEM_SHARED((N, d), jnp.float32),
            pltpu.SemaphoreType.DMA,
            pltpu.SemaphoreType.DMA,
            pltpu.SemaphoreType.DMA,
            pltpu.SemaphoreType.DMA,
            pltpu.SemaphoreType.DMA,
        ],
        compiler_params=pltpu.CompilerParams(use_tc_tiling_on_sc=False),
        name=name,
    )
    def agg_kernel(x_hbm, src_hbm, dst_hbm, out_hbm,
                   srcb, dstb, rows0, rows1, agg_sh,
                   isem, gsem0, gsem1, ssem0, ssem1):
        c = lax.axis_index("c")
        s = lax.axis_index("s")
        wid = s * ncores + c


        def zfill_row(i, _):
            def zfill_col(j, _):
                rows0[i, pl.ds(j * 16, 16)] = jnp.zeros((16,), jnp.float32)
                return ()
            return lax.fori_loop(0, d // 16, zfill_col, ())
        lax.fori_loop(0, min(CR, rows_per_tile), zfill_row, ())

        r0 = s * rows_per_tile
        left = rows_per_tile
        while left > 0:
            z = min(CR, left)
            pltpu.sync_copy(rows0.at[pl.ds(0, z)],
                            agg_sh.at[pl.ds(r0 + rows_per_tile - left, z)])
            left -= z
        plsc.subcore_barrier()

        c00 = wid * nchunk
        pltpu.sync_copy(src_hbm.at[c00], srcb.at[0])
        pltpu.sync_copy(dst_hbm.at[c00], dstb.at[0])
        gath0 = pltpu.async_copy(x_hbm.at[srcb.at[0]], rows0, gsem0)

        def pair(t, _):
            c0 = wid * nchunk + 2 * t

            @pl.when(t > 0)
            def _():
                pltpu.make_async_copy(rows1, agg_sh.at[dstb.at[1]],
                                      ssem1).wait()
            pltpu.sync_copy(src_hbm.at[c0 + 1], srcb.at[1])
            pltpu.sync_copy(dst_hbm.at[c0 + 1], dstb.at[1])
            g1 = pltpu.async_copy(x_hbm.at[srcb.at[1]], rows1, gsem1)
            pltpu.make_async_copy(x_hbm.at[srcb.at[0]], rows0, gsem0).wait()
            s0 = pltpu.async_copy(rows0, agg_sh.at[dstb.at[0]], ssem0,
                                  add=True)
            g1.wait()
            s1 = pltpu.async_copy(rows1, agg_sh.at[dstb.at[1]], ssem1,
                                  add=True)
            s0.wait()

            @pl.when(t < npair - 1)
            def _():
                pltpu.sync_copy(src_hbm.at[c0 + 2], srcb.at[0])
                pltpu.sync_copy(dst_hbm.at[c0 + 2], dstb.at[0])
                pltpu.async_copy(x_hbm.at[srcb.at[0]], rows0, gsem0)
            return ()
        lax.fori_loop(0, npair, pair, ())
        pltpu.make_async_copy(rows1, agg_sh.at[dstb.at[1]], ssem1).wait()
        plsc.subcore_barrier()

        pltpu.sync_copy(agg_sh.at[pl.ds(r0, rows_per_tile)],
                        out_hbm.at[c].at[pl.ds(r0, rows_per_tile)])

    return agg_kernel


def kernel(clinical, mel, edge_index, W_mel, b_mel, W_cat, b_cat,
           W1l, b1, W1r, W2l, b2, W2r):
    srcr = edge_index[0].reshape(-1, 125)
    dstr = edge_index[1].reshape(-1, 125)
    w2lp = jnp.pad(W2l, ((0, 0), (0, D2 - NC)))

    x_aug = _tc1(clinical, mel, W_mel, b_mel, W_cat[:CLIN], W_cat[CLIN:],
                 b_cat)
    agg1 = _make_sc_agg(D1, "sc_agg_l1")(x_aug, srcr, dstr)
    p2, r2 = _tc2(agg1, x_aug, W1l, b1, W1r, w2lp, W2r, b2)
    agg2 = _make_sc_agg(D2, "sc_agg_l2")(p2, srcr, dstr)
    return _tc3(agg2, r2)


# speedup vs baseline: 11.1181x; 1.2890x over previous
import functools

import jax
import jax.numpy as jnp
from jax import lax
from jax.experimental import pallas as pl
from jax.experimental.pallas import tpu as pltpu
from jax.experimental.pallas import tpu_sc as plsc

N = 10000
E = 320000
CLIN = 64
MEL = 128
HID = 128
NC = 4
D1 = 144
D2 = 16
BN = 1000
GRID = N // BN

_PREC = lax.Precision.HIGHEST


def _tc1_body(clin_ref, mel_ref, wm_ref, bm_ref, wc1_ref, wc2_ref, bc_ref,
              out_ref):
    mel_h = jnp.maximum(
        jnp.dot(mel_ref[...], wm_ref[...], precision=_PREC,
                preferred_element_type=jnp.float32) + bm_ref[...][None, :],
        0.0)
    xb = jnp.maximum(
        jnp.dot(clin_ref[...], wc1_ref[...], precision=_PREC,
                preferred_element_type=jnp.float32)
        + jnp.dot(mel_h, wc2_ref[...], precision=_PREC,
                  preferred_element_type=jnp.float32)
        + bc_ref[...][None, :],
        0.0)
    aug = (lax.broadcasted_iota(jnp.int32, (BN, D1 - HID), 1) == 0)
    out_ref[:, :HID] = xb
    out_ref[:, HID:] = aug.astype(jnp.float32)


def _tc2_body(agg_ref, xaug_ref, w1l_ref, b1_ref, w1r_ref, w2lp_ref, w2r_ref,
              b2_ref, p2_ref, r2_ref):
    aggs = agg_ref[0] + agg_ref[1]
    cnt = jnp.maximum(aggs[:, HID:HID + 1], 1.0)
    cinv = 1.0 / cnt
    mean1 = aggs[:, :HID] * cinv
    h1 = jnp.maximum(
        jnp.dot(mean1, w1l_ref[...], precision=_PREC,
                preferred_element_type=jnp.float32)
        + jnp.dot(xaug_ref[:, :HID], w1r_ref[...], precision=_PREC,
                  preferred_element_type=jnp.float32)
        + b1_ref[...][None, :],
        0.0)
    p2_ref[...] = jnp.dot(h1, w2lp_ref[...], precision=_PREC,
                          preferred_element_type=jnp.float32)
    r2 = jnp.dot(h1, w2r_ref[...], precision=_PREC,
                 preferred_element_type=jnp.float32) + b2_ref[...][None, :]
    r2_ref[...] = jnp.concatenate(
        [r2, cinv, jnp.zeros((BN, D2 - NC - 1), jnp.float32)], axis=1)


def _tc3_body(agg2_ref, r2_ref, out_ref):
    a = agg2_ref[0] + agg2_ref[1]
    out_ref[...] = a[:, :NC] * r2_ref[:, NC:NC + 1] + r2_ref[:, :NC]


def _full(shape):
    nd = len(shape)
    return pl.BlockSpec(shape, lambda i: (0,) * nd)


_tc1 = pl.pallas_call(
    _tc1_body,
    grid=(GRID,),
    in_specs=[
        pl.BlockSpec((BN, CLIN), lambda i: (i, 0)),
        pl.BlockSpec((BN, MEL), lambda i: (i, 0)),
        _full((MEL, HID)),
        _full((HID,)),
        _full((CLIN, HID)),
        _full((HID, HID)),
        _full((HID,)),
    ],
    out_specs=pl.BlockSpec((BN, D1), lambda i: (i, 0)),
    out_shape=jax.ShapeDtypeStruct((N, D1), jnp.float32),
)

_tc2 = pl.pallas_call(
    _tc2_body,
    grid=(GRID,),
    in_specs=[
        pl.BlockSpec((2, BN, D1), lambda i: (0, i, 0)),
        pl.BlockSpec((BN, D1), lambda i: (i, 0)),
        _full((HID, HID)),
        _full((HID,)),
        _full((HID, HID)),
        _full((HID, D2)),
        _full((HID, NC)),
        _full((NC,)),
    ],
    out_specs=[
        pl.BlockSpec((BN, D2), lambda i: (i, 0)),
        pl.BlockSpec((BN, D2), lambda i: (i, 0)),
    ],
    out_shape=[
        jax.ShapeDtypeStruct((N, D2), jnp.float32),
        jax.ShapeDtypeStruct((N, D2), jnp.float32),
    ],
)

_tc3 = pl.pallas_call(
    _tc3_body,
    grid=(GRID,),
    in_specs=[
        pl.BlockSpec((2, BN, D2), lambda i: (0, i, 0)),
        pl.BlockSpec((BN, D2), lambda i: (i, 0)),
    ],
    out_specs=pl.BlockSpec((BN, NC), lambda i: (i, 0)),
    out_shape=jax.ShapeDtypeStruct((N, NC), jnp.float32),
)


@functools.lru_cache(maxsize=None)
def _make_sc_agg(d, cr, depth, prefetch_all, name):
    info = plsc.get_sparse_core_info()
    ncores, nsub = info.num_cores, info.num_subcores
    nw = ncores * nsub
    ept = E // nw
    nchunk = ept // cr
    nround = nchunk // depth
    assert nchunk % depth == 0 and ept % cr == 0
    assert prefetch_all or nround % 2 == 0
    nisets = nchunk if prefetch_all else 2 * depth
    rows_per_tile = N // nsub
    mesh = plsc.VectorSubcoreMesh(core_axis_name="c", subcore_axis_name="s")

    @functools.partial(
        pl.kernel,
        out_type=jax.ShapeDtypeStruct((ncores, N, d), jnp.float32),
        mesh=mesh,
        scratch_types=[
            pltpu.VMEM((nisets, cr), jnp.int32),
            pltpu.VMEM((nisets, cr), jnp.int32),
            [pltpu.VMEM((cr, d), jnp.float32)] * depth,
            pltpu.VMEM_SHARED((N, d), jnp.float32),
            pltpu.SemaphoreType.DMA,
            [pltpu.SemaphoreType.DMA] * depth,
            [pltpu.SemaphoreType.DMA] * depth,
        ],
        compiler_params=pltpu.CompilerParams(use_tc_tiling_on_sc=False),
        name=name,
    )
    def agg_kernel(x_hbm, src_hbm, dst_hbm, out_hbm,
                   srcb, dstb, rows, agg_sh, isem, gsem, ssem):
        c = lax.axis_index("c")
        s = lax.axis_index("s")
        wid = s * ncores + c
        cbase = wid * nchunk

        def ifetch(slot, chunk):
            pltpu.async_copy(src_hbm.at[cbase + chunk], srcb.at[slot], isem)
            pltpu.async_copy(dst_hbm.at[cbase + chunk], dstb.at[slot], isem)

        def idrain_one():
            pltpu.make_async_copy(src_hbm.at[0], srcb.at[0], isem).wait()
            pltpu.make_async_copy(dst_hbm.at[0], dstb.at[0], isem).wait()

        if prefetch_all:
            def ifetch_loop(j, _):
                ifetch(j, j)
                return ()
            lax.fori_loop(0, nchunk, ifetch_loop, ())
        else:
            for b in range(depth):
                ifetch(b, b)

        def zfill_row(i, _):
            def zfill_col(j, _):
                rows[0][i, pl.ds(j * 16, 16)] = jnp.zeros((16,), jnp.float32)
                return ()
            return lax.fori_loop(0, d // 16, zfill_col, ())
        lax.fori_loop(0, min(cr, rows_per_tile), zfill_row, ())

        r0 = s * rows_per_tile
        left = rows_per_tile
        while left > 0:
            z = min(cr, left)
            pltpu.sync_copy(rows[0].at[pl.ds(0, z)],
                            agg_sh.at[pl.ds(r0 + rows_per_tile - left, z)])
            left -= z

        if prefetch_all:
            def idrain_loop(j, _):
                idrain_one()
                return ()
            lax.fori_loop(0, nchunk, idrain_loop, ())
        else:
            for _ in range(depth):
                idrain_one()
        plsc.subcore_barrier()

        def round_ops(t, islot0):
            j0 = t * depth
            for b in range(depth):
                @pl.when(t > 0)
                def _(b=b):
                    pltpu.make_async_copy(
                        rows[b], agg_sh.at[dstb.at[0]], ssem[b]).wait()
                if prefetch_all:
                    gidx = srcb.at[j0 + b]
                else:
                    gidx = srcb.at[islot0 + b]
                pltpu.async_copy(x_hbm.at[gidx], rows[b], gsem[b])
            if not prefetch_all:
                nslot0 = depth - islot0

                @pl.when(t + 1 < nround)
                def _():
                    for b in range(depth):
                        ifetch(nslot0 + b, j0 + depth + b)
            for b in range(depth):
                pltpu.make_async_copy(
                    x_hbm.at[srcb.at[0]], rows[b], gsem[b]).wait()
                if prefetch_all:
                    sidx = dstb.at[j0 + b]
                else:
                    sidx = dstb.at[islot0 + b]
                pltpu.async_copy(rows[b], agg_sh.at[sidx], ssem[b], add=True)
            if not prefetch_all:
                @pl.when(t + 1 < nround)
                def _():
                    for _ in range(depth):
                        idrain_one()

        if prefetch_all:
            def round_body(t, _):
                round_ops(t, 0)
                return ()
            lax.fori_loop(0, nround, round_body, ())
        else:
            def round_pair(u, _):
                round_ops(2 * u, 0)
                round_ops(2 * u + 1, depth)
                return ()
            lax.fori_loop(0, nround // 2, round_pair, ())
        for b in range(depth):
            pltpu.make_async_copy(rows[b], agg_sh.at[dstb.at[0]],
                                  ssem[b]).wait()
        plsc.subcore_barrier()

        pltpu.sync_copy(agg_sh.at[pl.ds(r0, rows_per_tile)],
                        out_hbm.at[c].at[pl.ds(r0, rows_per_tile)])

    return agg_kernel


def kernel(clinical, mel, edge_index, W_mel, b_mel, W_cat, b_cat,
           W1l, b1, W1r, W2l, b2, W2r):
    src1 = edge_index[0].reshape(-1, 50)
    dst1 = edge_index[1].reshape(-1, 50)
    src2 = edge_index[0].reshape(-1, 100)
    dst2 = edge_index[1].reshape(-1, 100)
    w2lp = jnp.pad(W2l, ((0, 0), (0, D2 - NC)))

    x_aug = _tc1(clinical, mel, W_mel, b_mel, W_cat[:CLIN], W_cat[CLIN:],
                 b_cat)
    agg1 = _make_sc_agg(D1, 50, 5, False, "sc_agg_l1")(x_aug, src1, dst1)
    p2, r2 = _tc2(agg1, x_aug, W1l, b1, W1r, w2lp, W2r, b2)
    agg2 = _make_sc_agg(D2, 100, 10, True, "sc_agg_l2")(p2, src2, dst2)
    return _tc3(agg2, r2)


# speedup vs baseline: 12.8086x; 1.1520x over previous
import functools

import jax
import jax.numpy as jnp
from jax import lax
from jax.experimental import pallas as pl
from jax.experimental.pallas import tpu as pltpu
from jax.experimental.pallas import tpu_sc as plsc

N = 10000
E = 320000
CLIN = 64
MEL = 128
HID = 128
NC = 4
D1 = 144
D2 = 16
BN = 2000
GRID = N // BN

_PREC = lax.Precision.DEFAULT


def _tc1_body(clin_ref, mel_ref, wm_ref, bm_ref, wc1_ref, wc2_ref, bc_ref,
              out_ref):
    mel_h = jnp.maximum(
        jnp.dot(mel_ref[...], wm_ref[...], precision=_PREC,
                preferred_element_type=jnp.float32) + bm_ref[...][None, :],
        0.0)
    xb = jnp.maximum(
        jnp.dot(clin_ref[...], wc1_ref[...], precision=_PREC,
                preferred_element_type=jnp.float32)
        + jnp.dot(mel_h, wc2_ref[...], precision=_PREC,
                  preferred_element_type=jnp.float32)
        + bc_ref[...][None, :],
        0.0)
    aug = (lax.broadcasted_iota(jnp.int32, (BN, D1 - HID), 1) == 0)
    out_ref[:, :HID] = xb
    out_ref[:, HID:] = aug.astype(jnp.float32)


def _tc2_body(agg_ref, xaug_ref, w1l_ref, b1_ref, w1r_ref, w2lp_ref, w2r_ref,
              b2_ref, p2_ref, r2_ref):
    aggs = agg_ref[0] + agg_ref[1]
    cnt = jnp.maximum(aggs[:, HID:HID + 1], 1.0)
    cinv = 1.0 / cnt
    mean1 = aggs[:, :HID] * cinv
    h1 = jnp.maximum(
        jnp.dot(mean1, w1l_ref[...], precision=_PREC,
                preferred_element_type=jnp.float32)
        + jnp.dot(xaug_ref[:, :HID], w1r_ref[...], precision=_PREC,
                  preferred_element_type=jnp.float32)
        + b1_ref[...][None, :],
        0.0)
    p2_ref[...] = jnp.dot(h1, w2lp_ref[...], precision=_PREC,
                          preferred_element_type=jnp.float32)
    r2 = jnp.dot(h1, w2r_ref[...], precision=_PREC,
                 preferred_element_type=jnp.float32) + b2_ref[...][None, :]
    r2_ref[...] = jnp.concatenate(
        [r2, cinv, jnp.zeros((BN, D2 - NC - 1), jnp.float32)], axis=1)


def _tc3_body(agg2_ref, r2_ref, out_ref):
    a = agg2_ref[0] + agg2_ref[1]
    out_ref[...] = a[:, :NC] * r2_ref[:, NC:NC + 1] + r2_ref[:, :NC]


def _full(shape):
    nd = len(shape)
    return pl.BlockSpec(shape, lambda i: (0,) * nd)


_tc1 = pl.pallas_call(
    _tc1_body,
    grid=(GRID,),
    in_specs=[
        pl.BlockSpec((BN, CLIN), lambda i: (i, 0)),
        pl.BlockSpec((BN, MEL), lambda i: (i, 0)),
        _full((MEL, HID)),
        _full((HID,)),
        _full((CLIN, HID)),
        _full((HID, HID)),
        _full((HID,)),
    ],
    out_specs=pl.BlockSpec((BN, D1), lambda i: (i, 0)),
    out_shape=jax.ShapeDtypeStruct((N, D1), jnp.float32),
)

_tc2 = pl.pallas_call(
    _tc2_body,
    grid=(GRID,),
    in_specs=[
        pl.BlockSpec((2, BN, D1), lambda i: (0, i, 0)),
        pl.BlockSpec((BN, D1), lambda i: (i, 0)),
        _full((HID, HID)),
        _full((HID,)),
        _full((HID, HID)),
        _full((HID, D2)),
        _full((HID, NC)),
        _full((NC,)),
    ],
    out_specs=[
        pl.BlockSpec((BN, D2), lambda i: (i, 0)),
        pl.BlockSpec((BN, D2), lambda i: (i, 0)),
    ],
    out_shape=[
        jax.ShapeDtypeStruct((N, D2), jnp.float32),
        jax.ShapeDtypeStruct((N, D2), jnp.float32),
    ],
)

_tc3 = pl.pallas_call(
    _tc3_body,
    grid=(GRID,),
    in_specs=[
        pl.BlockSpec((2, BN, D2), lambda i: (0, i, 0)),
        pl.BlockSpec((BN, D2), lambda i: (i, 0)),
    ],
    out_specs=pl.BlockSpec((BN, NC), lambda i: (i, 0)),
    out_shape=jax.ShapeDtypeStruct((N, NC), jnp.float32),
)


@functools.lru_cache(maxsize=None)
def _make_sc_agg(d, cr, depth, prefetch_all, name):
    info = plsc.get_sparse_core_info()
    ncores, nsub = info.num_cores, info.num_subcores
    nw = ncores * nsub
    ept = E // nw
    nchunk = ept // cr
    nround = nchunk // depth
    assert nchunk % depth == 0 and ept % cr == 0
    assert prefetch_all or nround % 2 == 0
    nisets = nchunk if prefetch_all else 2 * depth
    rows_per_tile = N // nsub
    mesh = plsc.VectorSubcoreMesh(core_axis_name="c", subcore_axis_name="s")

    @functools.partial(
        pl.kernel,
        out_type=jax.ShapeDtypeStruct((ncores, N, d), jnp.float32),
        mesh=mesh,
        scratch_types=[
            pltpu.VMEM((nisets, cr), jnp.int32),
            pltpu.VMEM((nisets, cr), jnp.int32),
            [pltpu.VMEM((cr, d), jnp.float32)] * depth,
            pltpu.VMEM_SHARED((N, d), jnp.float32),
            pltpu.SemaphoreType.DMA,
            [pltpu.SemaphoreType.DMA] * depth,
            [pltpu.SemaphoreType.DMA] * depth,
        ],
        compiler_params=pltpu.CompilerParams(use_tc_tiling_on_sc=False),
        name=name,
    )
    def agg_kernel(x_hbm, src_hbm, dst_hbm, out_hbm,
                   srcb, dstb, rows, agg_sh, isem, gsem, ssem):
        c = lax.axis_index("c")
        s = lax.axis_index("s")
        wid = s * ncores + c
        cbase = wid * nchunk

        def ifetch(slot, chunk):
            pltpu.async_copy(src_hbm.at[cbase + chunk], srcb.at[slot], isem)
            pltpu.async_copy(dst_hbm.at[cbase + chunk], dstb.at[slot], isem)

        def idrain_one():
            pltpu.make_async_copy(src_hbm.at[0], srcb.at[0], isem).wait()
            pltpu.make_async_copy(dst_hbm.at[0], dstb.at[0], isem).wait()

        if prefetch_all:
            def ifetch_loop(j, _):
                ifetch(j, j)
                return ()
            lax.fori_loop(0, nchunk, ifetch_loop, ())
        else:
            for b in range(depth):
                ifetch(b, b)

        def zfill_row(i, _):
            def zfill_col(j, _):
                rows[0][i, pl.ds(j * 16, 16)] = jnp.zeros((16,), jnp.float32)
                return ()
            return lax.fori_loop(0, d // 16, zfill_col, ())
        lax.fori_loop(0, min(cr, rows_per_tile), zfill_row, ())

        r0 = s * rows_per_tile
        left = rows_per_tile
        while left > 0:
            z = min(cr, left)
            pltpu.sync_copy(rows[0].at[pl.ds(0, z)],
                            agg_sh.at[pl.ds(r0 + rows_per_tile - left, z)])
            left -= z

        if prefetch_all:
            def idrain_loop(j, _):
                idrain_one()
                return ()
            lax.fori_loop(0, nchunk, idrain_loop, ())
        else:
            for _ in range(depth):
                idrain_one()
        plsc.subcore_barrier()

        def round_ops(t, islot0):
            j0 = t * depth
            for b in range(depth):
                @pl.when(t > 0)
                def _(b=b):
                    pltpu.make_async_copy(
                        rows[b], agg_sh.at[dstb.at[0]], ssem[b]).wait()
                if prefetch_all:
                    gidx = srcb.at[j0 + b]
                else:
                    gidx = srcb.at[islot0 + b]
                pltpu.async_copy(x_hbm.at[gidx], rows[b], gsem[b])
            if not prefetch_all:
                nslot0 = depth - islot0

                @pl.when(t + 1 < nround)
                def _():
                    for b in range(depth):
                        ifetch(nslot0 + b, j0 + depth + b)
            for b in range(depth):
                pltpu.make_async_copy(
                    x_hbm.at[srcb.at[0]], rows[b], gsem[b]).wait()
                if prefetch_all:
                    sidx = dstb.at[j0 + b]
                else:
                    sidx = dstb.at[islot0 + b]
                pltpu.async_copy(rows[b], agg_sh.at[sidx], ssem[b], add=True)
            if not prefetch_all:
                @pl.when(t + 1 < nround)
                def _():
                    for _ in range(depth):
                        idrain_one()

        if prefetch_all:
            def round_body(t, _):
                round_ops(t, 0)
                return ()
            lax.fori_loop(0, nround, round_body, ())
        else:
            def round_pair(u, _):
                round_ops(2 * u, 0)
                round_ops(2 * u + 1, depth)
                return ()
            lax.fori_loop(0, nround // 2, round_pair, ())
        for b in range(depth):
            pltpu.make_async_copy(rows[b], agg_sh.at[dstb.at[0]],
                                  ssem[b]).wait()
        plsc.subcore_barrier()

        pltpu.sync_copy(agg_sh.at[pl.ds(r0, rows_per_tile)],
                        out_hbm.at[c].at[pl.ds(r0, rows_per_tile)])

    return agg_kernel


def kernel(clinical, mel, edge_index, W_mel, b_mel, W_cat, b_cat,
           W1l, b1, W1r, W2l, b2, W2r):
    src1 = edge_index[0].reshape(-1, 50)
    dst1 = edge_index[1].reshape(-1, 50)
    w2lp = jnp.pad(W2l, ((0, 0), (0, D2 - NC)))

    x_aug = _tc1(clinical, mel, W_mel, b_mel, W_cat[:CLIN], W_cat[CLIN:],
                 b_cat)
    agg1 = _make_sc_agg(D1, 50, 5, False, "sc_agg_l1")(x_aug, src1, dst1)
    p2, r2 = _tc2(agg1, x_aug, W1l, b1, W1r, w2lp, W2r, b2)
    agg2 = _make_sc_agg(D2, 50, 10, True, "sc_agg_l2")(p2, src1, dst1)
    return _tc3(agg2, r2)


# speedup vs baseline: 13.2989x; 1.0383x over previous
import functools

import jax
import jax.numpy as jnp
from jax import lax
from jax.experimental import pallas as pl
from jax.experimental.pallas import tpu as pltpu
from jax.experimental.pallas import tpu_sc as plsc

N = 10000
E = 320000
CLIN = 64
MEL = 128
HID = 128
NC = 4
D1 = 144
D2 = 16
BN = 2000
GRID = N // BN

_PREC = lax.Precision.DEFAULT


def _tc1_body(clin_ref, mel_ref, wm_ref, bm_ref, wc1_ref, wc2_ref, bc_ref,
              out_ref):
    mel_h = jnp.maximum(
        jnp.dot(mel_ref[...], wm_ref[...], precision=_PREC,
                preferred_element_type=jnp.float32) + bm_ref[...][None, :],
        0.0)
    xb = jnp.maximum(
        jnp.dot(clin_ref[...], wc1_ref[...], precision=_PREC,
                preferred_element_type=jnp.float32)
        + jnp.dot(mel_h, wc2_ref[...], precision=_PREC,
                  preferred_element_type=jnp.float32)
        + bc_ref[...][None, :],
        0.0)
    aug = (lax.broadcasted_iota(jnp.int32, (BN, D1 - HID), 1) == 0)
    out_ref[:, :HID] = xb
    out_ref[:, HID:] = aug.astype(jnp.float32)


def _tc2_body(agg_ref, xaug_ref, w1l_ref, b1_ref, w1r_ref, w2lp_ref, w2r_ref,
              b2_ref, p2_ref, r2_ref):
    aggs = agg_ref[0] + agg_ref[1]
    cnt = jnp.maximum(aggs[:, HID:HID + 1], 1.0)
    cinv = 1.0 / cnt
    mean1 = aggs[:, :HID] * cinv
    h1 = jnp.maximum(
        jnp.dot(mean1, w1l_ref[...], precision=_PREC,
                preferred_element_type=jnp.float32)
        + jnp.dot(xaug_ref[:, :HID], w1r_ref[...], precision=_PREC,
                  preferred_element_type=jnp.float32)
        + b1_ref[...][None, :],
        0.0)
    p2_ref[:, :NC] = jnp.dot(h1, w2lp_ref[...], precision=_PREC,
                             preferred_element_type=jnp.float32)
    p2_ref[:, NC:] = jnp.zeros((BN, D2 - NC), jnp.float32)
    r2 = jnp.dot(h1, w2r_ref[...], precision=_PREC,
                 preferred_element_type=jnp.float32) + b2_ref[...][None, :]
    r2_ref[...] = jnp.concatenate(
        [r2, cinv, jnp.zeros((BN, D2 - NC - 1), jnp.float32)], axis=1)


def _tc3_body(agg2_ref, r2_ref, out_ref):
    a = agg2_ref[0] + agg2_ref[1]
    out_ref[...] = a[:, :NC] * r2_ref[:, NC:NC + 1] + r2_ref[:, :NC]


def _tc2_zero(shape):
    return jnp.zeros(shape, jnp.float32)


def _full(shape):
    nd = len(shape)
    return pl.BlockSpec(shape, lambda i: (0,) * nd)


_tc1 = pl.pallas_call(
    _tc1_body,
    grid=(GRID,),
    in_specs=[
        pl.BlockSpec((BN, CLIN), lambda i: (i, 0)),
        pl.BlockSpec((BN, MEL), lambda i: (i, 0)),
        _full((MEL, HID)),
        _full((HID,)),
        _full((CLIN, HID)),
        _full((HID, HID)),
        _full((HID,)),
    ],
    out_specs=pl.BlockSpec((BN, D1), lambda i: (i, 0)),
    out_shape=jax.ShapeDtypeStruct((N, D1), jnp.float32),
)

_tc2 = pl.pallas_call(
    _tc2_body,
    grid=(GRID,),
    in_specs=[
        pl.BlockSpec((2, BN, D1), lambda i: (0, i, 0)),
        pl.BlockSpec((BN, D1), lambda i: (i, 0)),
        _full((HID, HID)),
        _full((HID,)),
        _full((HID, HID)),
        _full((HID, NC)),
        _full((HID, NC)),
        _full((NC,)),
    ],
    out_specs=[
        pl.BlockSpec((BN, D2), lambda i: (i, 0)),
        pl.BlockSpec((BN, D2), lambda i: (i, 0)),
    ],
    out_shape=[
        jax.ShapeDtypeStruct((N, D2), jnp.float32),
        jax.ShapeDtypeStruct((N, D2), jnp.float32),
    ],
)

_tc3 = pl.pallas_call(
    _tc3_body,
    grid=(1,),
    in_specs=[
        pl.BlockSpec((2, N, D2), lambda i: (0, 0, 0)),
        pl.BlockSpec((N, D2), lambda i: (0, 0)),
    ],
    out_specs=pl.BlockSpec((N, NC), lambda i: (0, 0)),
    out_shape=jax.ShapeDtypeStruct((N, NC), jnp.float32),
)


@functools.lru_cache(maxsize=None)
def _make_sc_agg(d, cr, depth, prefetch_all, name):
    info = plsc.get_sparse_core_info()
    ncores, nsub = info.num_cores, info.num_subcores
    nw = ncores * nsub
    ept = E // nw
    nchunk = ept // cr
    nround = nchunk // depth
    assert nchunk % depth == 0 and ept % cr == 0
    assert prefetch_all or nround % 2 == 0
    nisets = nchunk if prefetch_all else 2 * depth
    rows_per_tile = N // nsub
    mesh = plsc.VectorSubcoreMesh(core_axis_name="c", subcore_axis_name="s")

    @functools.partial(
        pl.kernel,
        out_type=jax.ShapeDtypeStruct((ncores, N, d), jnp.float32),
        mesh=mesh,
        scratch_types=[
            pltpu.VMEM((nisets, cr), jnp.int32),
            pltpu.VMEM((nisets, cr), jnp.int32),
            [pltpu.VMEM((cr, d), jnp.float32)] * depth,
            pltpu.VMEM_SHARED((N, d), jnp.float32),
            pltpu.SemaphoreType.DMA,
            [pltpu.SemaphoreType.DMA] * depth,
            [pltpu.SemaphoreType.DMA] * depth,
        ],
        compiler_params=pltpu.CompilerParams(use_tc_tiling_on_sc=False),
        name=name,
    )
    def agg_kernel(x_hbm, edges_hbm, out_hbm,
                   srcb, dstb, rows, agg_sh, isem, gsem, ssem):
        c = lax.axis_index("c")
        s = lax.axis_index("s")
        wid = s * ncores + c
        cbase = wid * nchunk

        def ifetch(slot, chunk):
            pltpu.async_copy(edges_hbm.at[0, cbase + chunk], srcb.at[slot],
                             isem)
            pltpu.async_copy(edges_hbm.at[1, cbase + chunk], dstb.at[slot],
                             isem)

        def idrain_one():
            pltpu.make_async_copy(edges_hbm.at[0, 0], srcb.at[0], isem).wait()
            pltpu.make_async_copy(edges_hbm.at[0, 0], dstb.at[0], isem).wait()

        if prefetch_all:
            def ifetch_loop(j, _):
                ifetch(j, j)
                return ()
            lax.fori_loop(0, nchunk, ifetch_loop, ())
        else:
            for b in range(depth):
                ifetch(b, b)

        def zfill_row(i, _):
            def zfill_col(j, _):
                rows[0][i, pl.ds(j * 16, 16)] = jnp.zeros((16,), jnp.float32)
                return ()
            return lax.fori_loop(0, d // 16, zfill_col, ())
        lax.fori_loop(0, min(cr, rows_per_tile), zfill_row, ())

        r0 = s * rows_per_tile
        left = rows_per_tile
        while left > 0:
            z = min(cr, left)
            pltpu.sync_copy(rows[0].at[pl.ds(0, z)],
                            agg_sh.at[pl.ds(r0 + rows_per_tile - left, z)])
            left -= z

        if prefetch_all:
            def idrain_loop(j, _):
                idrain_one()
                return ()
            lax.fori_loop(0, nchunk, idrain_loop, ())
        else:
            for _ in range(depth):
                idrain_one()
        plsc.subcore_barrier()

        def round_ops(t, islot0):
            j0 = t * depth
            for b in range(depth):
                @pl.when(t > 0)
                def _(b=b):
                    pltpu.make_async_copy(
                        rows[b], agg_sh.at[dstb.at[0]], ssem[b]).wait()
                if prefetch_all:
                    gidx = srcb.at[j0 + b]
                else:
                    gidx = srcb.at[islot0 + b]
                pltpu.async_copy(x_hbm.at[gidx], rows[b], gsem[b])
            if not prefetch_all:
                nslot0 = depth - islot0

                @pl.when(t + 1 < nround)
                def _():
                    for b in range(depth):
                        ifetch(nslot0 + b, j0 + depth + b)
            for b in range(depth):
                pltpu.make_async_copy(
                    x_hbm.at[srcb.at[0]], rows[b], gsem[b]).wait()
                if prefetch_all:
                    sidx = dstb.at[j0 + b]
                else:
                    sidx = dstb.at[islot0 + b]
                pltpu.async_copy(rows[b], agg_sh.at[sidx], ssem[b], add=True)
            if not prefetch_all:
                @pl.when(t + 1 < nround)
                def _():
                    for _ in range(depth):
                        idrain_one()

        if prefetch_all:
            def round_body(t, _):
                round_ops(t, 0)
                return ()
            lax.fori_loop(0, nround, round_body, ())
        else:
            def round_pair(u, _):
                round_ops(2 * u, 0)
                round_ops(2 * u + 1, depth)
                return ()
            lax.fori_loop(0, nround // 2, round_pair, ())
        for b in range(depth):
            pltpu.make_async_copy(rows[b], agg_sh.at[dstb.at[0]],
                                  ssem[b]).wait()
        plsc.subcore_barrier()

        pltpu.sync_copy(agg_sh.at[pl.ds(r0, rows_per_tile)],
                        out_hbm.at[c].at[pl.ds(r0, rows_per_tile)])

    return agg_kernel


def kernel(clinical, mel, edge_index, W_mel, b_mel, W_cat, b_cat,
           W1l, b1, W1r, W2l, b2, W2r):
    edges = edge_index.reshape(2, -1, 50)

    x_aug = _tc1(clinical, mel, W_mel, b_mel, W_cat[:CLIN], W_cat[CLIN:],
                 b_cat)
    agg1 = _make_sc_agg(D1, 50, 5, False, "sc_agg_l1")(x_aug, edges)
    p2, r2 = _tc2(agg1, x_aug, W1l, b1, W1r, W2l, W2r, b2)
    agg2 = _make_sc_agg(D2, 50, 10, True, "sc_agg_l2")(p2, edges)
    return _tc3(agg2, r2)


# speedup vs baseline: 14.9307x; 1.1227x over previous
import functools

import jax
import jax.numpy as jnp
from jax import lax
from jax.experimental import pallas as pl
from jax.experimental.pallas import tpu as pltpu
from jax.experimental.pallas import tpu_sc as plsc

N = 10000
E = 320000
CLIN = 64
MEL = 128
HID = 128
NC = 4
D1 = 128
D2 = 16
BN = 2000
GRID = N // BN

_PREC = lax.Precision.DEFAULT


def _tc1_body(clin_ref, mel_ref, wm_ref, bm_ref, wc1_ref, wc2_ref, bc_ref,
              out_ref):
    mel_h = jnp.maximum(
        jnp.dot(mel_ref[...], wm_ref[...], precision=_PREC,
                preferred_element_type=jnp.float32) + bm_ref[...][None, :],
        0.0)
    xb = jnp.maximum(
        jnp.dot(clin_ref[...], wc1_ref[...], precision=_PREC,
                preferred_element_type=jnp.float32)
        + jnp.dot(mel_h, wc2_ref[...], precision=_PREC,
                  preferred_element_type=jnp.float32)
        + bc_ref[...][None, :],
        0.0)
    out_ref[...] = xb


def _tc2_body(agg_ref, cnt_ref, xaug_ref, w1l_ref, b1_ref, w1r_ref,
              w2lp_ref, w2r_ref, b2_ref, p2_ref, r2_ref):
    aggs = agg_ref[0] + agg_ref[1]
    cnt = jnp.maximum(cnt_ref[0][:, :1] + cnt_ref[1][:, :1], 1.0)
    cinv = 1.0 / cnt
    mean1 = aggs * cinv
    h1 = jnp.maximum(
        jnp.dot(mean1, w1l_ref[...], precision=_PREC,
                preferred_element_type=jnp.float32)
        + jnp.dot(xaug_ref[...], w1r_ref[...], precision=_PREC,
                  preferred_element_type=jnp.float32)
        + b1_ref[...][None, :],
        0.0)
    p2_ref[:, :NC] = jnp.dot(h1, w2lp_ref[...], precision=_PREC,
                             preferred_element_type=jnp.float32)
    p2_ref[:, NC:] = jnp.zeros((BN, D2 - NC), jnp.float32)
    r2 = jnp.dot(h1, w2r_ref[...], precision=_PREC,
                 preferred_element_type=jnp.float32) + b2_ref[...][None, :]
    r2_ref[...] = jnp.concatenate(
        [r2, cinv, jnp.zeros((BN, D2 - NC - 1), jnp.float32)], axis=1)


def _tc3_body(agg2_ref, r2_ref, out_ref):
    a = agg2_ref[0] + agg2_ref[1]
    out_ref[...] = a[:, :NC] * r2_ref[:, NC:NC + 1] + r2_ref[:, :NC]


def _tc2_zero(shape):
    return jnp.zeros(shape, jnp.float32)


def _full(shape):
    nd = len(shape)
    return pl.BlockSpec(shape, lambda i: (0,) * nd)


_tc1 = pl.pallas_call(
    _tc1_body,
    grid=(GRID,),
    in_specs=[
        pl.BlockSpec((BN, CLIN), lambda i: (i, 0)),
        pl.BlockSpec((BN, MEL), lambda i: (i, 0)),
        _full((MEL, HID)),
        _full((HID,)),
        _full((CLIN, HID)),
        _full((HID, HID)),
        _full((HID,)),
    ],
    out_specs=pl.BlockSpec((BN, D1), lambda i: (i, 0)),
    out_shape=jax.ShapeDtypeStruct((N, D1), jnp.float32),
)

_tc2 = pl.pallas_call(
    _tc2_body,
    grid=(GRID,),
    in_specs=[
        pl.BlockSpec((2, BN, D1), lambda i: (0, i, 0)),
        pl.BlockSpec((2, BN, D2), lambda i: (0, i, 0)),
        pl.BlockSpec((BN, D1), lambda i: (i, 0)),
        _full((HID, HID)),
        _full((HID,)),
        _full((HID, HID)),
        _full((HID, NC)),
        _full((HID, NC)),
        _full((NC,)),
    ],
    out_specs=[
        pl.BlockSpec((BN, D2), lambda i: (i, 0)),
        pl.BlockSpec((BN, D2), lambda i: (i, 0)),
    ],
    out_shape=[
        jax.ShapeDtypeStruct((N, D2), jnp.float32),
        jax.ShapeDtypeStruct((N, D2), jnp.float32),
    ],
)

_tc3 = pl.pallas_call(
    _tc3_body,
    grid=(1,),
    in_specs=[
        pl.BlockSpec((2, N, D2), lambda i: (0, 0, 0)),
        pl.BlockSpec((N, D2), lambda i: (0, 0)),
    ],
    out_specs=pl.BlockSpec((N, NC), lambda i: (0, 0)),
    out_shape=jax.ShapeDtypeStruct((N, NC), jnp.float32),
)


@functools.lru_cache(maxsize=None)
def _make_sc_agg(d, cr, depth, prefetch_all, with_cnt, name):
    info = plsc.get_sparse_core_info()
    ncores, nsub = info.num_cores, info.num_subcores
    nw = ncores * nsub
    ept = E // nw
    nchunk = ept // cr
    nround = nchunk // depth
    assert nchunk % depth == 0 and ept % cr == 0
    assert prefetch_all or nround % 2 == 0
    nisets = nchunk if prefetch_all else 2 * depth
    rows_per_tile = N // nsub
    mesh = plsc.VectorSubcoreMesh(core_axis_name="c", subcore_axis_name="s")

    out_type = jax.ShapeDtypeStruct((ncores, N, d), jnp.float32)
    if with_cnt:
        out_type = [out_type,
                    jax.ShapeDtypeStruct((ncores, N, 16), jnp.float32)]
    scratch = [
        pltpu.VMEM((nisets, cr), jnp.int32),
        pltpu.VMEM((nisets, cr), jnp.int32),
        [pltpu.VMEM((cr, d), jnp.float32)] * depth,
        pltpu.VMEM_SHARED((N, d), jnp.float32),
        pltpu.SemaphoreType.DMA,
        [pltpu.SemaphoreType.DMA] * depth,
        [pltpu.SemaphoreType.DMA] * depth,
    ]
    if with_cnt:
        scratch += [pltpu.VMEM_SHARED((N, 16), jnp.float32),
                    pltpu.VMEM((cr, 16), jnp.float32)]

    @functools.partial(
        pl.kernel,
        out_type=out_type,
        mesh=mesh,
        scratch_types=scratch,
        compiler_params=pltpu.CompilerParams(use_tc_tiling_on_sc=False),
        name=name,
    )
    def agg_kernel(x_hbm, edges_hbm, *out_and_scratch):
        if with_cnt:
            (out_hbm, cnt_hbm, srcb, dstb, rows, agg_sh, isem, gsem, ssem,
             cnt_sh, onesb) = out_and_scratch
        else:
            (out_hbm, srcb, dstb, rows, agg_sh, isem, gsem,
             ssem) = out_and_scratch
        c = lax.axis_index("c")
        s = lax.axis_index("s")
        wid = s * ncores + c
        cbase = wid * nchunk

        def ifetch(slot, chunk):
            pltpu.async_copy(edges_hbm.at[0, cbase + chunk], srcb.at[slot],
                             isem)
            pltpu.async_copy(edges_hbm.at[1, cbase + chunk], dstb.at[slot],
                             isem)

        def idrain_one():
            pltpu.make_async_copy(edges_hbm.at[0, 0], srcb.at[0], isem).wait()
            pltpu.make_async_copy(edges_hbm.at[0, 0], dstb.at[0], isem).wait()

        if prefetch_all:
            def ifetch_loop(j, _):
                ifetch(j, j)
                return ()
            lax.fori_loop(0, nchunk, ifetch_loop, ())
        else:
            for b in range(depth):
                ifetch(b, b)

        def zfill_row(i, _):
            def zfill_col(j, _):
                rows[0][i, pl.ds(j * 16, 16)] = jnp.zeros((16,), jnp.float32)
                return ()
            return lax.fori_loop(0, d // 16, zfill_col, ())
        lax.fori_loop(0, min(cr, rows_per_tile), zfill_row, ())

        r0 = s * rows_per_tile
        left = rows_per_tile
        while left > 0:
            z = min(cr, left)
            pltpu.sync_copy(rows[0].at[pl.ds(0, z)],
                            agg_sh.at[pl.ds(r0 + rows_per_tile - left, z)])
            left -= z
        if with_cnt:
            def onesfill(i, _):
                onesb[i, pl.ds(0, 16)] = jnp.ones((16,), jnp.float32)
                rows[1][i, pl.ds(0, 16)] = jnp.zeros((16,), jnp.float32)
                return ()
            lax.fori_loop(0, cr, onesfill, ())
            left = rows_per_tile
            while left > 0:
                z = min(cr, left)
                pltpu.sync_copy(
                    rows[1].at[pl.ds(0, z), pl.ds(0, 16)],
                    cnt_sh.at[pl.ds(r0 + rows_per_tile - left, z)])
                left -= z

        if prefetch_all:
            def idrain_loop(j, _):
                idrain_one()
                return ()
            lax.fori_loop(0, nchunk, idrain_loop, ())
        else:
            for _ in range(depth):
                idrain_one()
        plsc.subcore_barrier()

        def round_ops(t, islot0):
            j0 = t * depth
            for b in range(depth):
                @pl.when(t > 0)
                def _(b=b):
                    pltpu.make_async_copy(
                        rows[b], agg_sh.at[dstb.at[0]], ssem[b]).wait()
                    if with_cnt:
                        pltpu.make_async_copy(
                            onesb, cnt_sh.at[dstb.at[0]], ssem[b]).wait()
                if prefetch_all:
                    gidx = srcb.at[j0 + b]
                else:
                    gidx = srcb.at[islot0 + b]
                pltpu.async_copy(x_hbm.at[gidx], rows[b], gsem[b])
            if not prefetch_all:
                nslot0 = depth - islot0

                @pl.when(t + 1 < nround)
                def _():
                    for b in range(depth):
                        ifetch(nslot0 + b, j0 + depth + b)
            for b in range(depth):
                pltpu.make_async_copy(
                    x_hbm.at[srcb.at[0]], rows[b], gsem[b]).wait()
                if prefetch_all:
                    sidx = dstb.at[j0 + b]
                else:
                    sidx = dstb.at[islot0 + b]
                pltpu.async_copy(rows[b], agg_sh.at[sidx], ssem[b], add=True)
                if with_cnt:
                    pltpu.async_copy(onesb, cnt_sh.at[sidx], ssem[b],
                                     add=True)
            if not prefetch_all:
                @pl.when(t + 1 < nround)
                def _():
                    for _ in range(depth):
                        idrain_one()

        if prefetch_all:
            def round_body(t, _):
                round_ops(t, 0)
                return ()
            lax.fori_loop(0, nround, round_body, ())
        else:
            def round_pair(u, _):
                round_ops(2 * u, 0)
                round_ops(2 * u + 1, depth)
                return ()
            lax.fori_loop(0, nround // 2, round_pair, ())
        for b in range(depth):
            pltpu.make_async_copy(rows[b], agg_sh.at[dstb.at[0]],
                                  ssem[b]).wait()
            if with_cnt:
                pltpu.make_async_copy(onesb, cnt_sh.at[dstb.at[0]],
                                      ssem[b]).wait()
        plsc.subcore_barrier()

        pltpu.sync_copy(agg_sh.at[pl.ds(r0, rows_per_tile)],
                        out_hbm.at[c].at[pl.ds(r0, rows_per_tile)])
        if with_cnt:
            pltpu.sync_copy(cnt_sh.at[pl.ds(r0, rows_per_tile)],
                            cnt_hbm.at[c].at[pl.ds(r0, rows_per_tile)])

    return agg_kernel


def kernel(clinical, mel, edge_index, W_mel, b_mel, W_cat, b_cat,
           W1l, b1, W1r, W2l, b2, W2r):
    edges = edge_index.reshape(2, -1, 50)

    x_aug = _tc1(clinical, mel, W_mel, b_mel, W_cat[:CLIN], W_cat[CLIN:],
                 b_cat)
    agg1, cnt1 = _make_sc_agg(D1, 50, 5, False, True,
                              "sc_agg_l1")(x_aug, edges)
    p2, r2 = _tc2(agg1, cnt1, x_aug, W1l, b1, W1r, W2l, W2r, b2)
    agg2 = _make_sc_agg(D2, 50, 10, True, False, "sc_agg_l2")(p2, edges)
    return _tc3(agg2, r2)
